# Initial kernel scaffold; baseline (speedup 1.0000x reference)
import jax
import jax.numpy as jnp
from jax.experimental import pallas as pl


def kernel(x, coords, edge_index, params):
    raise NotImplementedError("write your pallas kernel here")


# speedup vs baseline: 1.6647x; 1.6647x over previous
import functools

import jax
import jax.numpy as jnp
from jax import lax
from jax.experimental import pallas as pl
from jax.experimental.pallas import tpu as pltpu
from jax.experimental.pallas import tpu_sc as plsc

_N = 10000
_E = 320000
_HID = 64
_F32 = jnp.float32

_NC, _NS, _L = 2, 16, 16
_NW = _NC * _NS
_EB = 128
_EPW = 10112
_GCHUNKS = _EPW // _EB
_E_PAD = _NW * _EPW
_EPT = _E_PAD // _NS
_ECHUNKS = _EPT // _EB
_N_PAD = 10240
_NHALF = _N_PAD // 2
_AROWS = 6144
_ARPT = _AROWS // _NS
_RPT = _N_PAD // _NS
_BE = 4096


def _silu(v):
    return v * jax.nn.sigmoid(v)


def _enc_body(x_ref, w1_ref, b1_ref, w2_ref, b2_ref, o_ref):
    h = _silu(jnp.dot(x_ref[...], w1_ref[...], preferred_element_type=_F32)
              + b1_ref[...])
    o_ref[...] = jnp.dot(h, w2_ref[...], preferred_element_type=_F32) + b2_ref[...]


def _hef_body(rel_ref, w_ref, w4_ref, b_ref, o_ref):
    r = rel_ref[...]
    d = jnp.sqrt(jnp.sum(r * r, axis=1, keepdims=True))
    pre = (jnp.dot(r, w_ref[...], preferred_element_type=_F32)
           + d * w4_ref[...] + b_ref[...])
    o_ref[...] = _silu(pre)


def _cmat_body(hef_ref, m_ref, o_ref):
    o_ref[...] = jnp.dot(hef_ref[...], m_ref[...], preferred_element_type=_F32)


def _pre_body(h_ref, wi_ref, wj_ref, we_ref, we2_ref, b0_ref, be2_ref,
              t_ref, m_ref):
    we = we_ref[...]
    m_ref[...] = jnp.dot(we2_ref[...], we, preferred_element_type=_F32)
    cvec = b0_ref[...] + jnp.dot(be2_ref[...], we, preferred_element_type=_F32)
    h = h_ref[...]
    a = jnp.dot(h, wi_ref[...], preferred_element_type=_F32) + cvec
    b = jnp.dot(h, wj_ref[...], preferred_element_type=_F32)
    t_ref[...] = jnp.concatenate([a, b], axis=1)


def _upd_body(acc_ref, h_ref, w2_ref, b2_ref,
              u1a_ref, u1b_ref, u1bias_ref, u2_ref, u2bias_ref,
              g_ref, gb_ref, o_ref):
    ss = jnp.concatenate([acc_ref[0, 0:_NHALF, 0:_HID],
                          acc_ref[1, 0:_NHALF, 0:_HID]], axis=0)
    cnt = jnp.concatenate([acc_ref[0, 0:_NHALF, _HID:_HID + 1],
                           acc_ref[1, 0:_NHALF, _HID:_HID + 1]], axis=0)
    invd = 1.0 / jnp.maximum(cnt, 1.0)
    alpha = jnp.minimum(cnt, 1.0)
    h = h_ref[...]
    agg = (jnp.dot(ss, w2_ref[...], preferred_element_type=_F32) * invd
           + alpha * b2_ref[...])
    t = _silu(jnp.dot(h, u1a_ref[...], preferred_element_type=_F32)
              + jnp.dot(agg, u1b_ref[...], preferred_element_type=_F32)
              + u1bias_ref[...])
    r = jnp.dot(t, u2_ref[...], preferred_element_type=_F32) + u2bias_ref[...] + h
    m = jnp.mean(r, axis=1, keepdims=True)
    v = jnp.mean(r * r, axis=1, keepdims=True) - m * m
    o_ref[...] = (r - m) * lax.rsqrt(v + 1e-5) * g_ref[...] + gb_ref[...]


_enc_call = pl.pallas_call(
    _enc_body, out_shape=jax.ShapeDtypeStruct((_N_PAD, _HID), _F32))

_dec_call = pl.pallas_call(
    _enc_body, out_shape=jax.ShapeDtypeStruct((_N_PAD, 16), _F32))

_hef_call = pl.pallas_call(
    _hef_body,
    grid=(_E_PAD // _BE,),
    in_specs=[pl.BlockSpec((_BE, 16), lambda i: (i, 0)),
              pl.BlockSpec((16, _HID), lambda i: (0, 0)),
              pl.BlockSpec((1, _HID), lambda i: (0, 0)),
              pl.BlockSpec((1, _HID), lambda i: (0, 0))],
    out_specs=pl.BlockSpec((_BE, _HID), lambda i: (i, 0)),
    out_shape=jax.ShapeDtypeStruct((_E_PAD, _HID), _F32))

_cmat_call = pl.pallas_call(
    _cmat_body,
    grid=(_E_PAD // _BE,),
    in_specs=[pl.BlockSpec((_BE, _HID), lambda i: (i, 0)),
              pl.BlockSpec((_HID, _HID), lambda i: (0, 0))],
    out_specs=pl.BlockSpec((_BE, _HID), lambda i: (i, 0)),
    out_shape=jax.ShapeDtypeStruct((_E_PAD, _HID), _F32))

_pre_call = pl.pallas_call(
    _pre_body,
    out_shape=[jax.ShapeDtypeStruct((_N_PAD, 2 * _HID), _F32),
               jax.ShapeDtypeStruct((_HID, _HID), _F32)])

_upd_call = pl.pallas_call(
    _upd_body, out_shape=jax.ShapeDtypeStruct((_N_PAD, _HID), _F32))


@functools.cache
def _sc_kernels():
    mesh = plsc.VectorSubcoreMesh(core_axis_name="c", subcore_axis_name="s",
                                  num_cores=_NC, num_subcores=_NS)

    @functools.partial(
        pl.kernel,
        out_type=jax.ShapeDtypeStruct((_E_PAD, 16), _F32),
        mesh=mesh,
        scratch_types=[
            pltpu.VMEM((_EB,), jnp.int32),
            pltpu.VMEM((_EB,), jnp.int32),
            pltpu.VMEM((_EB, 128), _F32),
            pltpu.VMEM((_EB, 128), _F32),
            pltpu.VMEM((_EB, 16), _F32),
            pltpu.SemaphoreType.DMA,
            pltpu.SemaphoreType.DMA,
        ])
    def _geom_kernel(coords_hbm, src_hbm, dst_hbm, rel_hbm,
                     dstv, srcv, cs, cd, rel16, s1, s2):
        cid = lax.axis_index("c")
        sid = lax.axis_index("s")
        wid = sid * _NC + cid

        def chunk(c, carry):
            base = wid * _EPW + c * _EB
            pltpu.sync_copy(dst_hbm.at[pl.ds(base, _EB)], dstv)
            pltpu.sync_copy(src_hbm.at[pl.ds(base, _EB)], srcv)
            cp1 = pltpu.async_copy(coords_hbm.at[srcv], cs, s1)
            cp2 = pltpu.async_copy(coords_hbm.at[dstv], cd, s2)
            cp1.wait()
            cp2.wait()

            def row(r, rc):
                rel16[r, :] = cd[r, pl.ds(0, 16)] - cs[r, pl.ds(0, 16)]
                return rc
            lax.fori_loop(0, _EB, row, 0)
            pltpu.sync_copy(rel16, rel_hbm.at[pl.ds(base, _EB)])
            return carry
        lax.fori_loop(0, _GCHUNKS, chunk, 0)

    @functools.partial(
        pl.kernel,
        out_type=jax.ShapeDtypeStruct((_E_PAD, 2 * _HID), _F32),
        mesh=mesh,
        scratch_types=[
            pltpu.VMEM((_EB,), jnp.int32),
            pltpu.VMEM((_EB,), jnp.int32),
            pltpu.VMEM((_EB, 2 * _HID), _F32),
            pltpu.VMEM((_EB, 2 * _HID), _F32),
            pltpu.VMEM((_EB, _HID), _F32),
            pltpu.VMEM((_EB, 2 * _HID), _F32),
            pltpu.SemaphoreType.DMA,
            pltpu.SemaphoreType.DMA,
        ])
    def _msg_kernel(t_hbm, cmat_hbm, src_hbm, dst_hbm, zeros_hbm,
                    s_hbm, dstv, srcv, td, ts, cm, sm, s1, s2):
        cid = lax.axis_index("c")
        sid = lax.axis_index("s")
        wid = sid * _NC + cid
        pltpu.sync_copy(zeros_hbm, sm)
        one0 = jnp.where(lax.iota(jnp.int32, _L) == 0, 1.0, 0.0).astype(_F32)

        def init_row(r, rc):
            sm[r, pl.ds(_HID, _L)] = one0
            return rc
        lax.fori_loop(0, _EB, init_row, 0)

        def chunk(c, carry):
            base = wid * _EPW + c * _EB
            pltpu.sync_copy(dst_hbm.at[pl.ds(base, _EB)], dstv)
            pltpu.sync_copy(src_hbm.at[pl.ds(base, _EB)], srcv)
            cp1 = pltpu.async_copy(t_hbm.at[dstv], td, s1)
            cp2 = pltpu.async_copy(t_hbm.at[srcv], ts, s2)
            pltpu.sync_copy(cmat_hbm.at[pl.ds(base, _EB)], cm)
            cp1.wait()
            cp2.wait()

            def row(r, rc):
                for kk in range(_HID // _L):
                    sl = pl.ds(kk * _L, _L)
                    sl2 = pl.ds(_HID + kk * _L, _L)
                    v = cm[r, sl] + td[r, sl] + ts[r, sl2]
                    sm[r, sl] = v / (1.0 + jnp.exp(-v))
                return rc
            lax.fori_loop(0, _EB, row, 0)
            pltpu.sync_copy(sm, s_hbm.at[pl.ds(base, _EB)])
            return carry
        lax.fori_loop(0, _GCHUNKS, chunk, 0)

    @functools.partial(
        pl.kernel,
        out_type=jax.ShapeDtypeStruct((_NC, _AROWS, 2 * _HID), _F32),
        mesh=mesh,
        scratch_types=[
            pltpu.VMEM((_EB,), jnp.int32),
            pltpu.VMEM((_EB, 2 * _HID), _F32),
            pltpu.VMEM((_EB, 2 * _HID), _F32),
            pltpu.VMEM_SHARED((_AROWS, 2 * _HID), _F32),
            pltpu.SemaphoreType.DMA,
        ])
    def _scat_kernel(s_hbm, dsth_hbm, zeros_hbm,
                     out_hbm, dstl, cm, zb, acc, s1):
        cid = lax.axis_index("c")
        sid = lax.axis_index("s")
        pltpu.sync_copy(zeros_hbm, zb)
        for t in range(_ARPT // _EB):
            pltpu.sync_copy(zb, acc.at[pl.ds(sid * _ARPT + t * _EB, _EB)])
        plsc.subcore_barrier()

        def chunk(c, carry):
            base = sid * _EPT + c * _EB
            pltpu.sync_copy(dsth_hbm.at[cid, pl.ds(base, _EB)], dstl)
            pltpu.sync_copy(s_hbm.at[pl.ds(base, _EB)], cm)
            pltpu.sync_copy(cm, acc.at[dstl], add=True)
            return carry
        lax.fori_loop(0, _ECHUNKS, chunk, 0)
        plsc.subcore_barrier()
        for t in range(_ARPT // _EB):
            r0 = sid * _ARPT + t * _EB
            pltpu.sync_copy(acc.at[pl.ds(r0, _EB)], cm)
            pltpu.sync_copy(cm, out_hbm.at[cid, pl.ds(r0, _EB)])

    return _geom_kernel, _msg_kernel, _scat_kernel


def kernel(x, coords, edge_index, params):
    p = params
    src = edge_index[0].astype(jnp.int32)
    dst = edge_index[1].astype(jnp.int32)
    srcp = jnp.concatenate([src, jnp.zeros((_E_PAD - _E,), jnp.int32)])
    dstp = jnp.concatenate([dst, jnp.full((_E_PAD - _E,), _N_PAD - 1, jnp.int32)])
    coords_p = jnp.pad(coords.astype(_F32), ((0, _N_PAD - _N), (0, 125)))
    xp = jnp.pad(x.astype(_F32), ((0, _N_PAD - _N), (0, 5)))

    enc1, enc2 = p["node_enc"]
    ee1, ee2 = p["edge_enc"]
    dec1, dec2 = p["dec"]

    dsth = jnp.stack([
        jnp.where(dstp < _NHALF, dstp, _AROWS - 1),
        jnp.where(dstp >= _NHALF, dstp - _NHALF, _AROWS - 1),
    ])
    zeros128 = jnp.zeros((_EB, 2 * _HID), _F32)

    _geom_kernel, _msg_kernel, _scat_kernel = _sc_kernels()
    relp = _geom_kernel(coords_p, srcp, dstp)

    we1p = jnp.pad(ee1["W"][:3], ((0, 13), (0, 0)))
    hef = _hef_call(relp, we1p, ee1["W"][3:4], ee1["b"].reshape(1, -1))

    w1p = jnp.pad(enc1["W"], ((0, 5), (0, 0)))
    h = _enc_call(xp, w1p, enc1["b"].reshape(1, -1),
                  enc2["W"], enc2["b"].reshape(1, -1))

    lyr = p["layers"]
    stk = (
        jnp.stack([l["edge_mlp"][0]["W"] for l in lyr]),
        jnp.stack([l["edge_mlp"][0]["b"].reshape(1, -1) for l in lyr]),
        jnp.stack([l["edge_mlp"][1]["W"] for l in lyr]),
        jnp.stack([l["edge_mlp"][1]["b"].reshape(1, -1) for l in lyr]),
        jnp.stack([l["node_mlp"][0]["W"] for l in lyr]),
        jnp.stack([l["node_mlp"][0]["b"].reshape(1, -1) for l in lyr]),
        jnp.stack([l["node_mlp"][1]["W"] for l in lyr]),
        jnp.stack([l["node_mlp"][1]["b"].reshape(1, -1) for l in lyr]),
        jnp.stack([l["ln_g"].reshape(1, -1) for l in lyr]),
        jnp.stack([l["ln_b"].reshape(1, -1) for l in lyr]),
    )
    we2 = ee2["W"]
    be2 = ee2["b"].reshape(1, -1)

    def _step(hc, w):
        em1w, em1b, em2w, em2b, nm1w, nm1b, nm2w, nm2b, lng, lnb = w
        t_t, m_t = _pre_call(hc, em1w[:_HID], em1w[_HID:2 * _HID],
                             em1w[2 * _HID:], we2, em1b, be2)
        cmat = _cmat_call(hef, m_t)
        s_t = _msg_kernel(t_t, cmat, srcp, dstp, zeros128)
        acc2 = _scat_kernel(s_t, dsth, zeros128)
        h2 = _upd_call(acc2, hc, em2w, em2b,
                       nm1w[:_HID], nm1w[_HID:], nm1b,
                       nm2w, nm2b, lng, lnb)
        return h2, jnp.float32(0)

    h, _ = lax.scan(_step, h, stk)

    d2p = jnp.pad(dec2["W"], ((0, 0), (0, 16 - 9)))
    d2bp = jnp.pad(dec2["b"], (0, 16 - 9)).reshape(1, -1)
    out16 = _dec_call(h, dec1["W"], dec1["b"].reshape(1, -1), d2p, d2bp)
    return out16[:_N, :9]


# speedup vs baseline: 1.8931x; 1.1372x over previous
import functools

import jax
import jax.numpy as jnp
from jax import lax
from jax.experimental import pallas as pl
from jax.experimental.pallas import tpu as pltpu
from jax.experimental.pallas import tpu_sc as plsc

_N = 10000
_E = 320000
_HID = 64
_F32 = jnp.float32

_NC, _NS, _L = 2, 16, 16
_NW = _NC * _NS
_EB = 128
_EPW = 10240
_GCHUNKS = _EPW // _EB
_E_PAD = _NW * _EPW
_EPT = _E_PAD // _NS
_ECHUNKS = _EPT // _EB
_N_PAD = 10240
_NHALF = _N_PAD // 2
_AROWS = 6144
_ARPT = _AROWS // _NS
_RPT = _N_PAD // _NS
_BE = 4096


def _silu(v):
    return v * jax.nn.sigmoid(v)


def _enc_body(x_ref, w1_ref, b1_ref, w2_ref, b2_ref, o_ref):
    h = _silu(jnp.dot(x_ref[...], w1_ref[...], preferred_element_type=_F32)
              + b1_ref[...])
    o_ref[...] = jnp.dot(h, w2_ref[...], preferred_element_type=_F32) + b2_ref[...]


def _hef_body(rel_ref, w_ref, w4_ref, b_ref, o_ref):
    r = rel_ref[...]
    d = jnp.sqrt(jnp.sum(r * r, axis=1, keepdims=True))
    pre = (jnp.dot(r, w_ref[...], preferred_element_type=_F32)
           + d * w4_ref[...] + b_ref[...])
    o_ref[...] = _silu(pre)


def _cmat_body(hef_ref, m_ref, o_ref):
    o_ref[...] = jnp.dot(hef_ref[...], m_ref[...], preferred_element_type=_F32)


def _pre_body(h_ref, wi_ref, wj_ref, we_ref, we2_ref, b0_ref, be2_ref,
              t_ref, m_ref):
    we = we_ref[...]
    m_ref[...] = jnp.dot(we2_ref[...], we, preferred_element_type=_F32)
    cvec = b0_ref[...] + jnp.dot(be2_ref[...], we, preferred_element_type=_F32)
    h = h_ref[...]
    a = jnp.dot(h, wi_ref[...], preferred_element_type=_F32) + cvec
    b = jnp.dot(h, wj_ref[...], preferred_element_type=_F32)
    t_ref[...] = jnp.concatenate([a, b], axis=1)


def _upd_body(acc_ref, h_ref, w2_ref, b2_ref,
              u1a_ref, u1b_ref, u1bias_ref, u2_ref, u2bias_ref,
              g_ref, gb_ref, o_ref):
    ss = jnp.concatenate([acc_ref[0, 0:_NHALF, 0:_HID],
                          acc_ref[1, 0:_NHALF, 0:_HID]], axis=0)
    cnt = jnp.concatenate([acc_ref[0, 0:_NHALF, _HID:_HID + 1],
                           acc_ref[1, 0:_NHALF, _HID:_HID + 1]], axis=0)
    invd = 1.0 / jnp.maximum(cnt, 1.0)
    alpha = jnp.minimum(cnt, 1.0)
    h = h_ref[...]
    agg = (jnp.dot(ss, w2_ref[...], preferred_element_type=_F32) * invd
           + alpha * b2_ref[...])
    t = _silu(jnp.dot(h, u1a_ref[...], preferred_element_type=_F32)
              + jnp.dot(agg, u1b_ref[...], preferred_element_type=_F32)
              + u1bias_ref[...])
    r = jnp.dot(t, u2_ref[...], preferred_element_type=_F32) + u2bias_ref[...] + h
    m = jnp.mean(r, axis=1, keepdims=True)
    v = jnp.mean(r * r, axis=1, keepdims=True) - m * m
    o_ref[...] = (r - m) * lax.rsqrt(v + 1e-5) * g_ref[...] + gb_ref[...]


_enc_call = pl.pallas_call(
    _enc_body, out_shape=jax.ShapeDtypeStruct((_N_PAD, _HID), _F32))

_dec_call = pl.pallas_call(
    _enc_body, out_shape=jax.ShapeDtypeStruct((_N_PAD, 16), _F32))

_hef_call = pl.pallas_call(
    _hef_body,
    grid=(_E_PAD // _BE,),
    in_specs=[pl.BlockSpec((_BE, 16), lambda i: (i, 0)),
              pl.BlockSpec((16, _HID), lambda i: (0, 0)),
              pl.BlockSpec((1, _HID), lambda i: (0, 0)),
              pl.BlockSpec((1, _HID), lambda i: (0, 0))],
    out_specs=pl.BlockSpec((_BE, _HID), lambda i: (i, 0)),
    out_shape=jax.ShapeDtypeStruct((_E_PAD, _HID), _F32))

_cmat_call = pl.pallas_call(
    _cmat_body,
    grid=(_E_PAD // _BE,),
    in_specs=[pl.BlockSpec((_BE, _HID), lambda i: (i, 0)),
              pl.BlockSpec((_HID, _HID), lambda i: (0, 0))],
    out_specs=pl.BlockSpec((_BE, _HID), lambda i: (i, 0)),
    out_shape=jax.ShapeDtypeStruct((_E_PAD, _HID), _F32))

_pre_call = pl.pallas_call(
    _pre_body,
    out_shape=[jax.ShapeDtypeStruct((_N_PAD, 2 * _HID), _F32),
               jax.ShapeDtypeStruct((_HID, _HID), _F32)])

_upd_call = pl.pallas_call(
    _upd_body, out_shape=jax.ShapeDtypeStruct((_N_PAD, _HID), _F32))


@functools.cache
def _sc_kernels():
    mesh = plsc.VectorSubcoreMesh(core_axis_name="c", subcore_axis_name="s",
                                  num_cores=_NC, num_subcores=_NS)

    @functools.partial(
        pl.kernel,
        out_type=jax.ShapeDtypeStruct((_E_PAD, 16), _F32),
        mesh=mesh,
        scratch_types=[
            pltpu.VMEM((_EB,), jnp.int32),
            pltpu.VMEM((_EB,), jnp.int32),
            pltpu.VMEM((_EB, 128), _F32),
            pltpu.VMEM((_EB, 128), _F32),
            pltpu.VMEM((_EB, 16), _F32),
            pltpu.SemaphoreType.DMA,
            pltpu.SemaphoreType.DMA,
        ])
    def _geom_kernel(coords_hbm, src_hbm, dst_hbm, rel_hbm,
                     dstv, srcv, cs, cd, rel16, s1, s2):
        cid = lax.axis_index("c")
        sid = lax.axis_index("s")
        wid = sid * _NC + cid

        def chunk(c, carry):
            base = wid * _EPW + c * _EB
            pltpu.sync_copy(dst_hbm.at[pl.ds(base, _EB)], dstv)
            pltpu.sync_copy(src_hbm.at[pl.ds(base, _EB)], srcv)
            cp1 = pltpu.async_copy(coords_hbm.at[srcv], cs, s1)
            cp2 = pltpu.async_copy(coords_hbm.at[dstv], cd, s2)
            cp1.wait()
            cp2.wait()

            def row(r, rc):
                rel16[r, :] = cd[r, pl.ds(0, 16)] - cs[r, pl.ds(0, 16)]
                return rc
            lax.fori_loop(0, _EB, row, 0)
            pltpu.sync_copy(rel16, rel_hbm.at[pl.ds(base, _EB)])
            return carry
        lax.fori_loop(0, _GCHUNKS, chunk, 0)

    @functools.partial(
        pl.kernel,
        out_type=jax.ShapeDtypeStruct((_E_PAD, 2 * _HID), _F32),
        mesh=mesh,
        scratch_types=[
            pltpu.VMEM((_EB,), jnp.int32),
            pltpu.VMEM((_EB,), jnp.int32),
            pltpu.VMEM((_EB,), jnp.int32),
            pltpu.VMEM((_EB,), jnp.int32),
            pltpu.VMEM((_EB, 2 * _HID), _F32),
            pltpu.VMEM((_EB, 2 * _HID), _F32),
            pltpu.VMEM((_EB, 2 * _HID), _F32),
            pltpu.VMEM((_EB, 2 * _HID), _F32),
            pltpu.VMEM((_EB, _HID), _F32),
            pltpu.VMEM((_EB, 2 * _HID), _F32),
            pltpu.VMEM((_EB, 2 * _HID), _F32),
            pltpu.SemaphoreType.DMA,
            pltpu.SemaphoreType.DMA,
            pltpu.SemaphoreType.DMA,
            pltpu.SemaphoreType.DMA,
            pltpu.SemaphoreType.DMA,
            pltpu.SemaphoreType.DMA,
            pltpu.SemaphoreType.DMA,
            pltpu.SemaphoreType.DMA,
        ])
    def _msg_kernel(t_hbm, cmat_hbm, src_hbm, dst_hbm, zeros_hbm, s_hbm,
                    dstv0, dstv1, srcv0, srcv1, td0, td1, ts0, ts1,
                    cm, sm0, sm1,
                    si0, si1, st0, st1, ss0, ss1, so0, so1):
        cid = lax.axis_index("c")
        sid = lax.axis_index("s")
        wid = sid * _NC + cid
        dstv = (dstv0, dstv1)
        srcv = (srcv0, srcv1)
        td = (td0, td1)
        ts = (ts0, ts1)
        sm = (sm0, sm1)
        si = (si0, si1)
        st = (st0, st1)
        ss = (ss0, ss1)
        so = (so0, so1)
        one0 = jnp.where(lax.iota(jnp.int32, _L) == 0, 1.0, 0.0).astype(_F32)
        for b in (0, 1):
            pltpu.sync_copy(zeros_hbm, sm[b])

            def init_row(r, rc, _b=b):
                sm[_b][r, pl.ds(_HID, _L)] = one0
                return rc
            lax.fori_loop(0, _EB, init_row, 0)

        base0 = wid * _EPW

        def _issue_idx(base, b):
            pltpu.async_copy(dst_hbm.at[pl.ds(base, _EB)], dstv[b], si[b])
            pltpu.async_copy(src_hbm.at[pl.ds(base, _EB)], srcv[b], si[b])

        def _wait_idx(b):
            pltpu.make_async_copy(dst_hbm.at[pl.ds(base0, _EB)], dstv[b],
                                  si[b]).wait()
            pltpu.make_async_copy(src_hbm.at[pl.ds(base0, _EB)], srcv[b],
                                  si[b]).wait()

        def _issue_gather(base, b):
            pltpu.async_copy(t_hbm.at[dstv[b]], td[b], st[b])
            pltpu.async_copy(t_hbm.at[srcv[b]], ts[b], ss[b])

        def _wait_gather(b):
            pltpu.make_async_copy(t_hbm.at[dstv[b]], td[b], st[b]).wait()
            pltpu.make_async_copy(t_hbm.at[srcv[b]], ts[b], ss[b]).wait()

        def _wait_store(b):
            pltpu.make_async_copy(sm[b], s_hbm.at[pl.ds(base0, _EB)],
                                  so[b]).wait()

        _issue_idx(base0, 0)
        _wait_idx(0)
        _issue_gather(base0, 0)
        _issue_idx(base0 + _EB, 1)
        pltpu.async_copy(sm[0], s_hbm.at[pl.ds(base0, _EB)], so[0])
        pltpu.async_copy(sm[1], s_hbm.at[pl.ds(base0 + _EB, _EB)], so[1])

        def pair(c2, carry):
            for b in (0, 1):
                idx = 2 * c2 + b
                nxt1 = jnp.where(idx + 1 == _GCHUNKS, 0, idx + 1)
                nxt2 = jnp.where(idx + 2 >= _GCHUNKS, idx + 2 - _GCHUNKS,
                                 idx + 2)
                _wait_idx(1 - b)
                _issue_gather(base0 + nxt1 * _EB, 1 - b)
                pltpu.sync_copy(cmat_hbm.at[pl.ds(base0 + idx * _EB, _EB)], cm)
                _wait_gather(b)
                _issue_idx(base0 + nxt2 * _EB, b)
                _wait_store(b)

                def row(r, rc, _b=b):
                    for kk in range(_HID // _L):
                        sl = pl.ds(kk * _L, _L)
                        sl2 = pl.ds(_HID + kk * _L, _L)
                        v = cm[r, sl] + td[_b][r, sl] + ts[_b][r, sl2]
                        sm[_b][r, sl] = v / (1.0 + jnp.exp(-v))
                    return rc
                lax.fori_loop(0, _EB, row, 0)
                pltpu.async_copy(sm[b], s_hbm.at[pl.ds(base0 + idx * _EB, _EB)],
                                 so[b])
            return carry
        lax.fori_loop(0, _GCHUNKS // 2, pair, 0)
        _wait_idx(1)
        _wait_gather(0)
        _wait_store(0)
        _wait_store(1)

    @functools.partial(
        pl.kernel,
        out_type=jax.ShapeDtypeStruct((_NC, _AROWS, 2 * _HID), _F32),
        mesh=mesh,
        scratch_types=[
            pltpu.VMEM((_EB,), jnp.int32),
            pltpu.VMEM((_EB,), jnp.int32),
            pltpu.VMEM((_EB, 2 * _HID), _F32),
            pltpu.VMEM((_EB, 2 * _HID), _F32),
            pltpu.VMEM((_EB, 2 * _HID), _F32),
            pltpu.VMEM_SHARED((_AROWS, 2 * _HID), _F32),
            pltpu.SemaphoreType.DMA,
            pltpu.SemaphoreType.DMA,
            pltpu.SemaphoreType.DMA,
            pltpu.SemaphoreType.DMA,
        ])
    def _scat_kernel(s_hbm, dsth_hbm, zeros_hbm,
                     out_hbm, dstl0, dstl1, cm0, cm1, zb, acc,
                     sd0, sd1, sc0, sc1):
        cid = lax.axis_index("c")
        sid = lax.axis_index("s")
        dstl = (dstl0, dstl1)
        cm = (cm0, cm1)
        sd = (sd0, sd1)
        sc = (sc0, sc1)
        pltpu.sync_copy(zeros_hbm, zb)
        for t in range(_ARPT // _EB):
            pltpu.sync_copy(zb, acc.at[pl.ds(sid * _ARPT + t * _EB, _EB)])
        plsc.subcore_barrier()

        base0 = sid * _EPT
        pltpu.async_copy(dsth_hbm.at[cid, pl.ds(base0, _EB)], dstl[0], sd[0])
        pltpu.async_copy(s_hbm.at[pl.ds(base0, _EB)], cm[0], sc[0])

        def pair(c2, carry):
            for b in (0, 1):
                idx = 2 * c2 + b
                pltpu.make_async_copy(
                    dsth_hbm.at[cid, pl.ds(base0, _EB)], dstl[b], sd[b]).wait()
                pltpu.make_async_copy(
                    s_hbm.at[pl.ds(base0, _EB)], cm[b], sc[b]).wait()
                nidx = jnp.where(idx + 1 == _ECHUNKS, 0, idx + 1)
                nbase = sid * _EPT + nidx * _EB
                pltpu.async_copy(dsth_hbm.at[cid, pl.ds(nbase, _EB)],
                                 dstl[1 - b], sd[1 - b])
                pltpu.async_copy(s_hbm.at[pl.ds(nbase, _EB)],
                                 cm[1 - b], sc[1 - b])
                pltpu.sync_copy(cm[b], acc.at[dstl[b]], add=True)
            return carry
        lax.fori_loop(0, _ECHUNKS // 2, pair, 0)
        pltpu.make_async_copy(
            dsth_hbm.at[cid, pl.ds(base0, _EB)], dstl[0], sd[0]).wait()
        pltpu.make_async_copy(
            s_hbm.at[pl.ds(base0, _EB)], cm[0], sc[0]).wait()
        plsc.subcore_barrier()
        for t in range(_ARPT // _EB):
            r0 = sid * _ARPT + t * _EB
            pltpu.sync_copy(acc.at[pl.ds(r0, _EB)], zb)
            pltpu.sync_copy(zb, out_hbm.at[cid, pl.ds(r0, _EB)])

    return _geom_kernel, _msg_kernel, _scat_kernel


def kernel(x, coords, edge_index, params):
    p = params
    src = edge_index[0].astype(jnp.int32)
    dst = edge_index[1].astype(jnp.int32)
    srcp = jnp.concatenate([src, jnp.zeros((_E_PAD - _E,), jnp.int32)])
    dstp = jnp.concatenate([dst, jnp.full((_E_PAD - _E,), _N_PAD - 1, jnp.int32)])
    coords_p = jnp.pad(coords.astype(_F32), ((0, _N_PAD - _N), (0, 125)))
    xp = jnp.pad(x.astype(_F32), ((0, _N_PAD - _N), (0, 5)))

    enc1, enc2 = p["node_enc"]
    ee1, ee2 = p["edge_enc"]
    dec1, dec2 = p["dec"]

    dsth = jnp.stack([
        jnp.where(dstp < _NHALF, dstp, _AROWS - 1),
        jnp.where(dstp >= _NHALF, dstp - _NHALF, _AROWS - 1),
    ])
    zeros128 = jnp.zeros((_EB, 2 * _HID), _F32)

    _geom_kernel, _msg_kernel, _scat_kernel = _sc_kernels()
    relp = _geom_kernel(coords_p, srcp, dstp)

    we1p = jnp.pad(ee1["W"][:3], ((0, 13), (0, 0)))
    hef = _hef_call(relp, we1p, ee1["W"][3:4], ee1["b"].reshape(1, -1))

    w1p = jnp.pad(enc1["W"], ((0, 5), (0, 0)))
    h = _enc_call(xp, w1p, enc1["b"].reshape(1, -1),
                  enc2["W"], enc2["b"].reshape(1, -1))

    lyr = p["layers"]
    stk = (
        jnp.stack([l["edge_mlp"][0]["W"] for l in lyr]),
        jnp.stack([l["edge_mlp"][0]["b"].reshape(1, -1) for l in lyr]),
        jnp.stack([l["edge_mlp"][1]["W"] for l in lyr]),
        jnp.stack([l["edge_mlp"][1]["b"].reshape(1, -1) for l in lyr]),
        jnp.stack([l["node_mlp"][0]["W"] for l in lyr]),
        jnp.stack([l["node_mlp"][0]["b"].reshape(1, -1) for l in lyr]),
        jnp.stack([l["node_mlp"][1]["W"] for l in lyr]),
        jnp.stack([l["node_mlp"][1]["b"].reshape(1, -1) for l in lyr]),
        jnp.stack([l["ln_g"].reshape(1, -1) for l in lyr]),
        jnp.stack([l["ln_b"].reshape(1, -1) for l in lyr]),
    )
    we2 = ee2["W"]
    be2 = ee2["b"].reshape(1, -1)

    def _step(hc, w):
        em1w, em1b, em2w, em2b, nm1w, nm1b, nm2w, nm2b, lng, lnb = w
        t_t, m_t = _pre_call(hc, em1w[:_HID], em1w[_HID:2 * _HID],
                             em1w[2 * _HID:], we2, em1b, be2)
        cmat = _cmat_call(hef, m_t)
        s_t = _msg_kernel(t_t, cmat, srcp, dstp, zeros128)
        acc2 = _scat_kernel(s_t, dsth, zeros128)
        h2 = _upd_call(acc2, hc, em2w, em2b,
                       nm1w[:_HID], nm1w[_HID:], nm1b,
                       nm2w, nm2b, lng, lnb)
        return h2, jnp.float32(0)

    h, _ = lax.scan(_step, h, stk)

    d2p = jnp.pad(dec2["W"], ((0, 0), (0, 16 - 9)))
    d2bp = jnp.pad(dec2["b"], (0, 16 - 9)).reshape(1, -1)
    out16 = _dec_call(h, dec1["W"], dec1["b"].reshape(1, -1), d2p, d2bp)
    return out16[:_N, :9]


# speedup vs baseline: 1.8985x; 1.0029x over previous
import functools

import jax
import jax.numpy as jnp
from jax import lax
from jax.experimental import pallas as pl
from jax.experimental.pallas import tpu as pltpu
from jax.experimental.pallas import tpu_sc as plsc

_N = 10000
_E = 320000
_HID = 64
_F32 = jnp.float32

_NC, _NS, _L = 2, 16, 16
_NW = _NC * _NS
_EB = 128
_EPW = 10240
_GCHUNKS = _EPW // _EB
_E_PAD = _NW * _EPW
_EPT = _E_PAD // _NS
_ECHUNKS = _EPT // _EB
_N_PAD = 10240
_NHALF = _N_PAD // 2
_AROWS = 6144
_ARPT = _AROWS // _NS
_RPT = _N_PAD // _NS
_BE = 4096


def _silu(v):
    return v * jax.nn.sigmoid(v)


def _enc_body(x_ref, w1_ref, b1_ref, w2_ref, b2_ref, o_ref):
    h = _silu(jnp.dot(x_ref[...], w1_ref[...], preferred_element_type=_F32)
              + b1_ref[...])
    o_ref[...] = jnp.dot(h, w2_ref[...], preferred_element_type=_F32) + b2_ref[...]


def _hef_body(rel_ref, w_ref, w4_ref, b_ref, o_ref):
    r = rel_ref[...]
    d = jnp.sqrt(jnp.sum(r * r, axis=1, keepdims=True))
    pre = (jnp.dot(r, w_ref[...], preferred_element_type=_F32)
           + d * w4_ref[...] + b_ref[...])
    o_ref[...] = _silu(pre)


def _cmat_body(hef_ref, m_ref, o_ref):
    o_ref[...] = jnp.dot(hef_ref[...], m_ref[...], preferred_element_type=_F32)


def _pre_body(h_ref, wi_ref, wj_ref, we_ref, we2_ref, b0_ref, be2_ref,
              t_ref, m_ref):
    we = we_ref[...]
    m_ref[...] = jnp.dot(we2_ref[...], we, preferred_element_type=_F32)
    cvec = b0_ref[...] + jnp.dot(be2_ref[...], we, preferred_element_type=_F32)
    h = h_ref[...]
    a = jnp.dot(h, wi_ref[...], preferred_element_type=_F32) + cvec
    b = jnp.dot(h, wj_ref[...], preferred_element_type=_F32)
    t_ref[...] = jnp.concatenate([a, b], axis=1)


def _upd_body(acc_ref, h_ref, w2_ref, b2_ref,
              u1a_ref, u1b_ref, u1bias_ref, u2_ref, u2bias_ref,
              g_ref, gb_ref, o_ref):
    ss = jnp.concatenate([acc_ref[0, 0:_NHALF, 0:_HID],
                          acc_ref[1, 0:_NHALF, 0:_HID]], axis=0)
    cnt = jnp.concatenate([acc_ref[0, 0:_NHALF, _HID:_HID + 1],
                           acc_ref[1, 0:_NHALF, _HID:_HID + 1]], axis=0)
    invd = 1.0 / jnp.maximum(cnt, 1.0)
    alpha = jnp.minimum(cnt, 1.0)
    h = h_ref[...]
    agg = (jnp.dot(ss, w2_ref[...], preferred_element_type=_F32) * invd
           + alpha * b2_ref[...])
    t = _silu(jnp.dot(h, u1a_ref[...], preferred_element_type=_F32)
              + jnp.dot(agg, u1b_ref[...], preferred_element_type=_F32)
              + u1bias_ref[...])
    r = jnp.dot(t, u2_ref[...], preferred_element_type=_F32) + u2bias_ref[...] + h
    m = jnp.mean(r, axis=1, keepdims=True)
    v = jnp.mean(r * r, axis=1, keepdims=True) - m * m
    o_ref[...] = (r - m) * lax.rsqrt(v + 1e-5) * g_ref[...] + gb_ref[...]


_enc_call = pl.pallas_call(
    _enc_body, out_shape=jax.ShapeDtypeStruct((_N_PAD, _HID), _F32))

_dec_call = pl.pallas_call(
    _enc_body, out_shape=jax.ShapeDtypeStruct((_N_PAD, 16), _F32))

_hef_call = pl.pallas_call(
    _hef_body,
    grid=(_E_PAD // _BE,),
    in_specs=[pl.BlockSpec((_BE, 16), lambda i: (i, 0)),
              pl.BlockSpec((16, _HID), lambda i: (0, 0)),
              pl.BlockSpec((1, _HID), lambda i: (0, 0)),
              pl.BlockSpec((1, _HID), lambda i: (0, 0))],
    out_specs=pl.BlockSpec((_BE, _HID), lambda i: (i, 0)),
    out_shape=jax.ShapeDtypeStruct((_E_PAD, _HID), _F32))

_cmat_call = pl.pallas_call(
    _cmat_body,
    grid=(_E_PAD // _BE,),
    in_specs=[pl.BlockSpec((_BE, _HID), lambda i: (i, 0)),
              pl.BlockSpec((_HID, _HID), lambda i: (0, 0))],
    out_specs=pl.BlockSpec((_BE, _HID), lambda i: (i, 0)),
    out_shape=jax.ShapeDtypeStruct((_E_PAD, _HID), _F32))

_pre_call = pl.pallas_call(
    _pre_body,
    out_shape=[jax.ShapeDtypeStruct((_N_PAD, 2 * _HID), _F32),
               jax.ShapeDtypeStruct((_HID, _HID), _F32)])

_upd_call = pl.pallas_call(
    _upd_body, out_shape=jax.ShapeDtypeStruct((_N_PAD, _HID), _F32))


@functools.cache
def _sc_kernels():
    mesh = plsc.VectorSubcoreMesh(core_axis_name="c", subcore_axis_name="s",
                                  num_cores=_NC, num_subcores=_NS)

    @functools.partial(
        pl.kernel,
        out_type=jax.ShapeDtypeStruct((_E_PAD, 16), _F32),
        mesh=mesh,
        scratch_types=[
            pltpu.VMEM((_EB,), jnp.int32),
            pltpu.VMEM((_EB,), jnp.int32),
            pltpu.VMEM((_EB, 128), _F32),
            pltpu.VMEM((_EB, 128), _F32),
            pltpu.VMEM((_EB, 16), _F32),
            pltpu.SemaphoreType.DMA,
            pltpu.SemaphoreType.DMA,
        ])
    def _geom_kernel(coords_hbm, src_hbm, dst_hbm, rel_hbm,
                     dstv, srcv, cs, cd, rel16, s1, s2):
        cid = lax.axis_index("c")
        sid = lax.axis_index("s")
        wid = sid * _NC + cid

        def chunk(c, carry):
            base = wid * _EPW + c * _EB
            pltpu.sync_copy(dst_hbm.at[pl.ds(base, _EB)], dstv)
            pltpu.sync_copy(src_hbm.at[pl.ds(base, _EB)], srcv)
            cp1 = pltpu.async_copy(coords_hbm.at[srcv], cs, s1)
            cp2 = pltpu.async_copy(coords_hbm.at[dstv], cd, s2)
            cp1.wait()
            cp2.wait()

            def row(r, rc):
                rel16[r, :] = cd[r, pl.ds(0, 16)] - cs[r, pl.ds(0, 16)]
                return rc
            lax.fori_loop(0, _EB, row, 0)
            pltpu.sync_copy(rel16, rel_hbm.at[pl.ds(base, _EB)])
            return carry
        lax.fori_loop(0, _GCHUNKS, chunk, 0)

    @functools.partial(
        pl.kernel,
        out_type=jax.ShapeDtypeStruct((_E_PAD, 2 * _HID), _F32),
        mesh=mesh,
        scratch_types=[
            pltpu.VMEM((_EB,), jnp.int32),
            pltpu.VMEM((_EB,), jnp.int32),
            pltpu.VMEM((_EB,), jnp.int32),
            pltpu.VMEM((_EB,), jnp.int32),
            pltpu.VMEM((_EB, 2 * _HID), _F32),
            pltpu.VMEM((_EB, 2 * _HID), _F32),
            pltpu.VMEM((_EB, 2 * _HID), _F32),
            pltpu.VMEM((_EB, 2 * _HID), _F32),
            pltpu.VMEM((_EB, _HID), _F32),
            pltpu.VMEM((_EB, 2 * _HID), _F32),
            pltpu.VMEM((_EB, 2 * _HID), _F32),
            pltpu.SemaphoreType.DMA,
            pltpu.SemaphoreType.DMA,
            pltpu.SemaphoreType.DMA,
            pltpu.SemaphoreType.DMA,
            pltpu.SemaphoreType.DMA,
            pltpu.SemaphoreType.DMA,
            pltpu.SemaphoreType.DMA,
            pltpu.SemaphoreType.DMA,
        ])
    def _msg_kernel(t_hbm, cmat_hbm, src_hbm, dst_hbm, zeros_hbm, s_hbm,
                    dstv0, dstv1, srcv0, srcv1, td0, td1, ts0, ts1,
                    cm, sm0, sm1,
                    si0, si1, st0, st1, ss0, ss1, so0, so1):
        cid = lax.axis_index("c")
        sid = lax.axis_index("s")
        wid = sid * _NC + cid
        dstv = (dstv0, dstv1)
        srcv = (srcv0, srcv1)
        td = (td0, td1)
        ts = (ts0, ts1)
        sm = (sm0, sm1)
        si = (si0, si1)
        st = (st0, st1)
        ss = (ss0, ss1)
        so = (so0, so1)
        one0 = jnp.where(lax.iota(jnp.int32, _L) == 0, 1.0, 0.0).astype(_F32)
        for b in (0, 1):
            pltpu.sync_copy(zeros_hbm, sm[b])

            def init_row(r, rc, _b=b):
                sm[_b][r, pl.ds(_HID, _L)] = one0
                return rc
            lax.fori_loop(0, _EB, init_row, 0)

        base0 = wid * _EPW

        def _issue_idx(base, b):
            pltpu.async_copy(dst_hbm.at[pl.ds(base, _EB)], dstv[b], si[b])
            pltpu.async_copy(src_hbm.at[pl.ds(base, _EB)], srcv[b], si[b])

        def _wait_idx(b):
            pltpu.make_async_copy(dst_hbm.at[pl.ds(base0, _EB)], dstv[b],
                                  si[b]).wait()
            pltpu.make_async_copy(src_hbm.at[pl.ds(base0, _EB)], srcv[b],
                                  si[b]).wait()

        def _issue_gather(base, b):
            pltpu.async_copy(t_hbm.at[dstv[b]], td[b], st[b])
            pltpu.async_copy(t_hbm.at[srcv[b]], ts[b], ss[b])

        def _wait_gather(b):
            pltpu.make_async_copy(t_hbm.at[dstv[b]], td[b], st[b]).wait()
            pltpu.make_async_copy(t_hbm.at[srcv[b]], ts[b], ss[b]).wait()

        def _wait_store(b):
            pltpu.make_async_copy(sm[b], s_hbm.at[pl.ds(base0, _EB)],
                                  so[b]).wait()

        _issue_idx(base0, 0)
        _wait_idx(0)
        _issue_gather(base0, 0)
        _issue_idx(base0 + _EB, 1)
        pltpu.async_copy(sm[0], s_hbm.at[pl.ds(base0, _EB)], so[0])
        pltpu.async_copy(sm[1], s_hbm.at[pl.ds(base0 + _EB, _EB)], so[1])

        def pair(c2, carry):
            for b in (0, 1):
                idx = 2 * c2 + b
                nxt1 = jnp.where(idx + 1 == _GCHUNKS, 0, idx + 1)
                nxt2 = jnp.where(idx + 2 >= _GCHUNKS, idx + 2 - _GCHUNKS,
                                 idx + 2)
                _wait_idx(1 - b)
                _issue_gather(base0 + nxt1 * _EB, 1 - b)
                pltpu.sync_copy(cmat_hbm.at[pl.ds(base0 + idx * _EB, _EB)], cm)
                _wait_gather(b)
                _issue_idx(base0 + nxt2 * _EB, b)
                _wait_store(b)

                @plsc.parallel_loop(0, _EB, unroll=4)
                def _rows(r, _b=b):
                    for kk in range(_HID // _L):
                        sl = pl.ds(kk * _L, _L)
                        sl2 = pl.ds(_HID + kk * _L, _L)
                        v = cm[r, sl] + td[_b][r, sl] + ts[_b][r, sl2]
                        sm[_b][r, sl] = v / (1.0 + jnp.exp(-v))
                pltpu.async_copy(sm[b], s_hbm.at[pl.ds(base0 + idx * _EB, _EB)],
                                 so[b])
            return carry
        lax.fori_loop(0, _GCHUNKS // 2, pair, 0)
        _wait_idx(1)
        _wait_gather(0)
        _wait_store(0)
        _wait_store(1)

    @functools.partial(
        pl.kernel,
        out_type=jax.ShapeDtypeStruct((_NC, _AROWS, 2 * _HID), _F32),
        mesh=mesh,
        scratch_types=[
            pltpu.VMEM((_EB,), jnp.int32),
            pltpu.VMEM((_EB,), jnp.int32),
            pltpu.VMEM((_EB, 2 * _HID), _F32),
            pltpu.VMEM((_EB, 2 * _HID), _F32),
            pltpu.VMEM((_EB, 2 * _HID), _F32),
            pltpu.VMEM_SHARED((_AROWS, 2 * _HID), _F32),
            pltpu.SemaphoreType.DMA,
            pltpu.SemaphoreType.DMA,
            pltpu.SemaphoreType.DMA,
            pltpu.SemaphoreType.DMA,
        ])
    def _scat_kernel(s_hbm, dsth_hbm, zeros_hbm,
                     out_hbm, dstl0, dstl1, cm0, cm1, zb, acc,
                     sd0, sd1, sc0, sc1):
        cid = lax.axis_index("c")
        sid = lax.axis_index("s")
        dstl = (dstl0, dstl1)
        cm = (cm0, cm1)
        sd = (sd0, sd1)
        sc = (sc0, sc1)
        pltpu.sync_copy(zeros_hbm, zb)
        for t in range(_ARPT // _EB):
            pltpu.sync_copy(zb, acc.at[pl.ds(sid * _ARPT + t * _EB, _EB)])
        plsc.subcore_barrier()

        base0 = sid * _EPT
        pltpu.async_copy(dsth_hbm.at[cid, pl.ds(base0, _EB)], dstl[0], sd[0])
        pltpu.async_copy(s_hbm.at[pl.ds(base0, _EB)], cm[0], sc[0])

        def pair(c2, carry):
            for b in (0, 1):
                idx = 2 * c2 + b
                pltpu.make_async_copy(
                    dsth_hbm.at[cid, pl.ds(base0, _EB)], dstl[b], sd[b]).wait()
                pltpu.make_async_copy(
                    s_hbm.at[pl.ds(base0, _EB)], cm[b], sc[b]).wait()
                nidx = jnp.where(idx + 1 == _ECHUNKS, 0, idx + 1)
                nbase = sid * _EPT + nidx * _EB
                pltpu.async_copy(dsth_hbm.at[cid, pl.ds(nbase, _EB)],
                                 dstl[1 - b], sd[1 - b])
                pltpu.async_copy(s_hbm.at[pl.ds(nbase, _EB)],
                                 cm[1 - b], sc[1 - b])
                pltpu.sync_copy(cm[b], acc.at[dstl[b]], add=True)
            return carry
        lax.fori_loop(0, _ECHUNKS // 2, pair, 0)
        pltpu.make_async_copy(
            dsth_hbm.at[cid, pl.ds(base0, _EB)], dstl[0], sd[0]).wait()
        pltpu.make_async_copy(
            s_hbm.at[pl.ds(base0, _EB)], cm[0], sc[0]).wait()
        plsc.subcore_barrier()
        for t in range(_ARPT // _EB):
            r0 = sid * _ARPT + t * _EB
            pltpu.sync_copy(acc.at[pl.ds(r0, _EB)], zb)
            pltpu.sync_copy(zb, out_hbm.at[cid, pl.ds(r0, _EB)])

    return _geom_kernel, _msg_kernel, _scat_kernel


def kernel(x, coords, edge_index, params):
    p = params
    src = edge_index[0].astype(jnp.int32)
    dst = edge_index[1].astype(jnp.int32)
    srcp = jnp.concatenate([src, jnp.zeros((_E_PAD - _E,), jnp.int32)])
    dstp = jnp.concatenate([dst, jnp.full((_E_PAD - _E,), _N_PAD - 1, jnp.int32)])
    coords_p = jnp.pad(coords.astype(_F32), ((0, _N_PAD - _N), (0, 125)))
    xp = jnp.pad(x.astype(_F32), ((0, _N_PAD - _N), (0, 5)))

    enc1, enc2 = p["node_enc"]
    ee1, ee2 = p["edge_enc"]
    dec1, dec2 = p["dec"]

    dsth = jnp.stack([
        jnp.where(dstp < _NHALF, dstp, _AROWS - 1),
        jnp.where(dstp >= _NHALF, dstp - _NHALF, _AROWS - 1),
    ])
    zeros128 = jnp.zeros((_EB, 2 * _HID), _F32)

    _geom_kernel, _msg_kernel, _scat_kernel = _sc_kernels()
    relp = _geom_kernel(coords_p, srcp, dstp)

    we1p = jnp.pad(ee1["W"][:3], ((0, 13), (0, 0)))
    hef = _hef_call(relp, we1p, ee1["W"][3:4], ee1["b"].reshape(1, -1))

    w1p = jnp.pad(enc1["W"], ((0, 5), (0, 0)))
    h = _enc_call(xp, w1p, enc1["b"].reshape(1, -1),
                  enc2["W"], enc2["b"].reshape(1, -1))

    lyr = p["layers"]
    stk = (
        jnp.stack([l["edge_mlp"][0]["W"] for l in lyr]),
        jnp.stack([l["edge_mlp"][0]["b"].reshape(1, -1) for l in lyr]),
        jnp.stack([l["edge_mlp"][1]["W"] for l in lyr]),
        jnp.stack([l["edge_mlp"][1]["b"].reshape(1, -1) for l in lyr]),
        jnp.stack([l["node_mlp"][0]["W"] for l in lyr]),
        jnp.stack([l["node_mlp"][0]["b"].reshape(1, -1) for l in lyr]),
        jnp.stack([l["node_mlp"][1]["W"] for l in lyr]),
        jnp.stack([l["node_mlp"][1]["b"].reshape(1, -1) for l in lyr]),
        jnp.stack([l["ln_g"].reshape(1, -1) for l in lyr]),
        jnp.stack([l["ln_b"].reshape(1, -1) for l in lyr]),
    )
    we2 = ee2["W"]
    be2 = ee2["b"].reshape(1, -1)

    def _step(hc, w):
        em1w, em1b, em2w, em2b, nm1w, nm1b, nm2w, nm2b, lng, lnb = w
        t_t, m_t = _pre_call(hc, em1w[:_HID], em1w[_HID:2 * _HID],
                             em1w[2 * _HID:], we2, em1b, be2)
        cmat = _cmat_call(hef, m_t)
        s_t = _msg_kernel(t_t, cmat, srcp, dstp, zeros128)
        acc2 = _scat_kernel(s_t, dsth, zeros128)
        h2 = _upd_call(acc2, hc, em2w, em2b,
                       nm1w[:_HID], nm1w[_HID:], nm1b,
                       nm2w, nm2b, lng, lnb)
        return h2, jnp.float32(0)

    h, _ = lax.scan(_step, h, stk)

    d2p = jnp.pad(dec2["W"], ((0, 0), (0, 16 - 9)))
    d2bp = jnp.pad(dec2["b"], (0, 16 - 9)).reshape(1, -1)
    out16 = _dec_call(h, dec1["W"], dec1["b"].reshape(1, -1), d2p, d2bp)
    return out16[:_N, :9]


# speedup vs baseline: 1.9419x; 1.0229x over previous
import functools

import jax
import jax.numpy as jnp
from jax import lax
from jax.experimental import pallas as pl
from jax.experimental.pallas import tpu as pltpu
from jax.experimental.pallas import tpu_sc as plsc

_N = 10000
_E = 320000
_HID = 64
_F32 = jnp.float32

_NC, _NS, _L = 2, 16, 16
_NW = _NC * _NS
_EB = 128
_EPW = 10240
_GCHUNKS = _EPW // _EB
_E_PAD = _NW * _EPW
_EPT = _E_PAD // _NS
_ECHUNKS = _EPT // _EB
_N_PAD = 10240
_NHALF = _N_PAD // 2
_AROWS = 6144
_ARPT = _AROWS // _NS
_RPT = _N_PAD // _NS
_BE = 4096


def _silu(v):
    return v * jax.nn.sigmoid(v)


def _enc_body(x_ref, w1_ref, b1_ref, w2_ref, b2_ref, o_ref):
    h = _silu(jnp.dot(x_ref[...], w1_ref[...], preferred_element_type=_F32)
              + b1_ref[...])
    o_ref[...] = jnp.dot(h, w2_ref[...], preferred_element_type=_F32) + b2_ref[...]


def _hef_body(rel_ref, w_ref, w4_ref, b_ref, o_ref):
    r = rel_ref[...]
    d = jnp.sqrt(jnp.sum(r * r, axis=1, keepdims=True))
    pre = (jnp.dot(r, w_ref[...], preferred_element_type=_F32)
           + d * w4_ref[...] + b_ref[...])
    o_ref[...] = _silu(pre)


def _cmat_body(hef_ref, m_ref, o_ref):
    o_ref[...] = jnp.dot(hef_ref[...], m_ref[...], preferred_element_type=_F32)


def _pre_body(h_ref, wi_ref, wj_ref, we_ref, we2_ref, b0_ref, be2_ref,
              t_ref, m_ref):
    we = we_ref[...]
    m_ref[...] = jnp.dot(we2_ref[...], we, preferred_element_type=_F32)
    cvec = b0_ref[...] + jnp.dot(be2_ref[...], we, preferred_element_type=_F32)
    h = h_ref[...]
    a = jnp.dot(h, wi_ref[...], preferred_element_type=_F32) + cvec
    b = jnp.dot(h, wj_ref[...], preferred_element_type=_F32)
    t_ref[...] = jnp.concatenate([a, b], axis=1)


def _upd_body(acc_ref, h_ref, w2_ref, b2_ref,
              u1a_ref, u1b_ref, u1bias_ref, u2_ref, u2bias_ref,
              g_ref, gb_ref, o_ref):
    ss = jnp.concatenate([acc_ref[0, 0:_NHALF, 0:_HID],
                          acc_ref[1, 0:_NHALF, 0:_HID]], axis=0)
    cnt = jnp.concatenate([acc_ref[0, 0:_NHALF, _HID:_HID + 1],
                           acc_ref[1, 0:_NHALF, _HID:_HID + 1]], axis=0)
    invd = 1.0 / jnp.maximum(cnt, 1.0)
    alpha = jnp.minimum(cnt, 1.0)
    h = h_ref[...]
    agg = (jnp.dot(ss, w2_ref[...], preferred_element_type=_F32) * invd
           + alpha * b2_ref[...])
    t = _silu(jnp.dot(h, u1a_ref[...], preferred_element_type=_F32)
              + jnp.dot(agg, u1b_ref[...], preferred_element_type=_F32)
              + u1bias_ref[...])
    r = jnp.dot(t, u2_ref[...], preferred_element_type=_F32) + u2bias_ref[...] + h
    m = jnp.mean(r, axis=1, keepdims=True)
    v = jnp.mean(r * r, axis=1, keepdims=True) - m * m
    o_ref[...] = (r - m) * lax.rsqrt(v + 1e-5) * g_ref[...] + gb_ref[...]


_enc_call = pl.pallas_call(
    _enc_body, out_shape=jax.ShapeDtypeStruct((_N_PAD, _HID), _F32))

_dec_call = pl.pallas_call(
    _enc_body, out_shape=jax.ShapeDtypeStruct((_N_PAD, 16), _F32))

_hef_call = pl.pallas_call(
    _hef_body,
    grid=(_E_PAD // _BE,),
    in_specs=[pl.BlockSpec((_BE, 16), lambda i: (i, 0)),
              pl.BlockSpec((16, _HID), lambda i: (0, 0)),
              pl.BlockSpec((1, _HID), lambda i: (0, 0)),
              pl.BlockSpec((1, _HID), lambda i: (0, 0))],
    out_specs=pl.BlockSpec((_BE, _HID), lambda i: (i, 0)),
    out_shape=jax.ShapeDtypeStruct((_E_PAD, _HID), _F32))

_cmat_call = pl.pallas_call(
    _cmat_body,
    grid=(_E_PAD // _BE,),
    in_specs=[pl.BlockSpec((_BE, _HID), lambda i: (i, 0)),
              pl.BlockSpec((_HID, _HID), lambda i: (0, 0))],
    out_specs=pl.BlockSpec((_BE, _HID), lambda i: (i, 0)),
    out_shape=jax.ShapeDtypeStruct((_E_PAD, _HID), _F32))

_pre_call = pl.pallas_call(
    _pre_body,
    out_shape=[jax.ShapeDtypeStruct((_N_PAD, 2 * _HID), _F32),
               jax.ShapeDtypeStruct((_HID, _HID), _F32)])

_upd_call = pl.pallas_call(
    _upd_body, out_shape=jax.ShapeDtypeStruct((_N_PAD, _HID), _F32))


@functools.cache
def _sc_kernels():
    mesh = plsc.VectorSubcoreMesh(core_axis_name="c", subcore_axis_name="s",
                                  num_cores=_NC, num_subcores=_NS)

    @functools.partial(
        pl.kernel,
        out_type=jax.ShapeDtypeStruct((_E_PAD, 16), _F32),
        mesh=mesh,
        scratch_types=[
            pltpu.VMEM((_EB,), jnp.int32),
            pltpu.VMEM((_EB,), jnp.int32),
            pltpu.VMEM((_EB,), jnp.int32),
            pltpu.VMEM((_EB,), jnp.int32),
            pltpu.VMEM((_EB, 128), _F32),
            pltpu.VMEM((_EB, 128), _F32),
            pltpu.VMEM((_EB, 128), _F32),
            pltpu.VMEM((_EB, 128), _F32),
            pltpu.VMEM((_EB, 16), _F32),
            pltpu.VMEM((_EB, 16), _F32),
            pltpu.SemaphoreType.DMA,
            pltpu.SemaphoreType.DMA,
            pltpu.SemaphoreType.DMA,
            pltpu.SemaphoreType.DMA,
            pltpu.SemaphoreType.DMA,
            pltpu.SemaphoreType.DMA,
            pltpu.SemaphoreType.DMA,
            pltpu.SemaphoreType.DMA,
        ])
    def _geom_kernel(coords_hbm, src_hbm, dst_hbm, rel_hbm,
                     dstv0, dstv1, srcv0, srcv1, cs0, cs1, cd0, cd1,
                     rel0, rel1, si0, si1, st0, st1, ss0, ss1, so0, so1):
        cid = lax.axis_index("c")
        sid = lax.axis_index("s")
        wid = sid * _NC + cid
        dstv = (dstv0, dstv1)
        srcv = (srcv0, srcv1)
        cs = (cs0, cs1)
        cd = (cd0, cd1)
        rel16 = (rel0, rel1)
        si = (si0, si1)
        st = (st0, st1)
        ss = (ss0, ss1)
        so = (so0, so1)
        base0 = wid * _EPW

        def _issue_idx(base, b):
            pltpu.async_copy(dst_hbm.at[pl.ds(base, _EB)], dstv[b], si[b])
            pltpu.async_copy(src_hbm.at[pl.ds(base, _EB)], srcv[b], si[b])

        def _wait_idx(b):
            pltpu.make_async_copy(dst_hbm.at[pl.ds(base0, _EB)], dstv[b],
                                  si[b]).wait()
            pltpu.make_async_copy(src_hbm.at[pl.ds(base0, _EB)], srcv[b],
                                  si[b]).wait()

        def _issue_gather(b):
            pltpu.async_copy(coords_hbm.at[srcv[b]], cs[b], st[b])
            pltpu.async_copy(coords_hbm.at[dstv[b]], cd[b], ss[b])

        def _wait_gather(b):
            pltpu.make_async_copy(coords_hbm.at[srcv[b]], cs[b], st[b]).wait()
            pltpu.make_async_copy(coords_hbm.at[dstv[b]], cd[b], ss[b]).wait()

        def _wait_store(b):
            pltpu.make_async_copy(rel16[b], rel_hbm.at[pl.ds(base0, _EB)],
                                  so[b]).wait()

        _issue_idx(base0, 0)
        _wait_idx(0)
        _issue_gather(0)
        _issue_idx(base0 + _EB, 1)
        pltpu.async_copy(rel16[0], rel_hbm.at[pl.ds(base0, _EB)], so[0])
        pltpu.async_copy(rel16[1], rel_hbm.at[pl.ds(base0 + _EB, _EB)], so[1])

        def pair(c2, carry):
            for b in (0, 1):
                idx = 2 * c2 + b
                nxt2 = jnp.where(idx + 2 >= _GCHUNKS, idx + 2 - _GCHUNKS,
                                 idx + 2)
                _wait_idx(1 - b)
                _issue_gather(1 - b)
                _wait_gather(b)
                _issue_idx(base0 + nxt2 * _EB, b)
                _wait_store(b)

                @plsc.parallel_loop(0, _EB, unroll=4)
                def _rows(r, _b=b):
                    rel16[_b][r, :] = (cd[_b][r, pl.ds(0, 16)]
                                       - cs[_b][r, pl.ds(0, 16)])
                pltpu.async_copy(rel16[b],
                                 rel_hbm.at[pl.ds(base0 + idx * _EB, _EB)],
                                 so[b])
            return carry
        lax.fori_loop(0, _GCHUNKS // 2, pair, 0)
        _wait_idx(1)
        _wait_gather(0)
        _wait_store(0)
        _wait_store(1)

    @functools.partial(
        pl.kernel,
        out_type=jax.ShapeDtypeStruct((_E_PAD, 2 * _HID), _F32),
        mesh=mesh,
        scratch_types=[
            pltpu.VMEM((_EB,), jnp.int32),
            pltpu.VMEM((_EB,), jnp.int32),
            pltpu.VMEM((_EB,), jnp.int32),
            pltpu.VMEM((_EB,), jnp.int32),
            pltpu.VMEM((_EB, 2 * _HID), _F32),
            pltpu.VMEM((_EB, 2 * _HID), _F32),
            pltpu.VMEM((_EB, 2 * _HID), _F32),
            pltpu.VMEM((_EB, 2 * _HID), _F32),
            pltpu.VMEM((_EB, _HID), _F32),
            pltpu.VMEM((_EB, 2 * _HID), _F32),
            pltpu.VMEM((_EB, 2 * _HID), _F32),
            pltpu.SemaphoreType.DMA,
            pltpu.SemaphoreType.DMA,
            pltpu.SemaphoreType.DMA,
            pltpu.SemaphoreType.DMA,
            pltpu.SemaphoreType.DMA,
            pltpu.SemaphoreType.DMA,
            pltpu.SemaphoreType.DMA,
            pltpu.SemaphoreType.DMA,
        ])
    def _msg_kernel(t_hbm, cmat_hbm, src_hbm, dst_hbm, zeros_hbm, s_hbm,
                    dstv0, dstv1, srcv0, srcv1, td0, td1, ts0, ts1,
                    cm, sm0, sm1,
                    si0, si1, st0, st1, ss0, ss1, so0, so1):
        cid = lax.axis_index("c")
        sid = lax.axis_index("s")
        wid = sid * _NC + cid
        dstv = (dstv0, dstv1)
        srcv = (srcv0, srcv1)
        td = (td0, td1)
        ts = (ts0, ts1)
        sm = (sm0, sm1)
        si = (si0, si1)
        st = (st0, st1)
        ss = (ss0, ss1)
        so = (so0, so1)
        one0 = jnp.where(lax.iota(jnp.int32, _L) == 0, 1.0, 0.0).astype(_F32)
        for b in (0, 1):
            pltpu.sync_copy(zeros_hbm, sm[b])

            def init_row(r, rc, _b=b):
                sm[_b][r, pl.ds(_HID, _L)] = one0
                return rc
            lax.fori_loop(0, _EB, init_row, 0)

        base0 = wid * _EPW

        def _issue_idx(base, b):
            pltpu.async_copy(dst_hbm.at[pl.ds(base, _EB)], dstv[b], si[b])
            pltpu.async_copy(src_hbm.at[pl.ds(base, _EB)], srcv[b], si[b])

        def _wait_idx(b):
            pltpu.make_async_copy(dst_hbm.at[pl.ds(base0, _EB)], dstv[b],
                                  si[b]).wait()
            pltpu.make_async_copy(src_hbm.at[pl.ds(base0, _EB)], srcv[b],
                                  si[b]).wait()

        def _issue_gather(base, b):
            pltpu.async_copy(t_hbm.at[dstv[b]], td[b], st[b])
            pltpu.async_copy(t_hbm.at[srcv[b]], ts[b], ss[b])

        def _wait_gather(b):
            pltpu.make_async_copy(t_hbm.at[dstv[b]], td[b], st[b]).wait()
            pltpu.make_async_copy(t_hbm.at[srcv[b]], ts[b], ss[b]).wait()

        def _wait_store(b):
            pltpu.make_async_copy(sm[b], s_hbm.at[pl.ds(base0, _EB)],
                                  so[b]).wait()

        _issue_idx(base0, 0)
        _wait_idx(0)
        _issue_gather(base0, 0)
        _issue_idx(base0 + _EB, 1)
        pltpu.async_copy(sm[0], s_hbm.at[pl.ds(base0, _EB)], so[0])
        pltpu.async_copy(sm[1], s_hbm.at[pl.ds(base0 + _EB, _EB)], so[1])

        def pair(c2, carry):
            for b in (0, 1):
                idx = 2 * c2 + b
                nxt1 = jnp.where(idx + 1 == _GCHUNKS, 0, idx + 1)
                nxt2 = jnp.where(idx + 2 >= _GCHUNKS, idx + 2 - _GCHUNKS,
                                 idx + 2)
                _wait_idx(1 - b)
                _issue_gather(base0 + nxt1 * _EB, 1 - b)
                pltpu.sync_copy(cmat_hbm.at[pl.ds(base0 + idx * _EB, _EB)], cm)
                _wait_gather(b)
                _issue_idx(base0 + nxt2 * _EB, b)
                _wait_store(b)

                @plsc.parallel_loop(0, _EB, unroll=4)
                def _rows(r, _b=b):
                    for kk in range(_HID // _L):
                        sl = pl.ds(kk * _L, _L)
                        sl2 = pl.ds(_HID + kk * _L, _L)
                        v = cm[r, sl] + td[_b][r, sl] + ts[_b][r, sl2]
                        sm[_b][r, sl] = v / (1.0 + jnp.exp(-v))
                pltpu.async_copy(sm[b], s_hbm.at[pl.ds(base0 + idx * _EB, _EB)],
                                 so[b])
            return carry
        lax.fori_loop(0, _GCHUNKS // 2, pair, 0)
        _wait_idx(1)
        _wait_gather(0)
        _wait_store(0)
        _wait_store(1)

    @functools.partial(
        pl.kernel,
        out_type=jax.ShapeDtypeStruct((_NC, _AROWS, 2 * _HID), _F32),
        mesh=mesh,
        scratch_types=[
            pltpu.VMEM((_EB,), jnp.int32),
            pltpu.VMEM((_EB,), jnp.int32),
            pltpu.VMEM((_EB, 2 * _HID), _F32),
            pltpu.VMEM((_EB, 2 * _HID), _F32),
            pltpu.VMEM((_EB, 2 * _HID), _F32),
            pltpu.VMEM_SHARED((_AROWS, 2 * _HID), _F32),
            pltpu.SemaphoreType.DMA,
            pltpu.SemaphoreType.DMA,
            pltpu.SemaphoreType.DMA,
            pltpu.SemaphoreType.DMA,
        ])
    def _scat_kernel(s_hbm, dsth_hbm, zeros_hbm,
                     out_hbm, dstl0, dstl1, cm0, cm1, zb, acc,
                     sd0, sd1, sc0, sc1):
        cid = lax.axis_index("c")
        sid = lax.axis_index("s")
        dstl = (dstl0, dstl1)
        cm = (cm0, cm1)
        sd = (sd0, sd1)
        sc = (sc0, sc1)
        pltpu.sync_copy(zeros_hbm, zb)
        for t in range(_ARPT // _EB):
            pltpu.sync_copy(zb, acc.at[pl.ds(sid * _ARPT + t * _EB, _EB)])
        plsc.subcore_barrier()

        base0 = sid * _EPT
        pltpu.async_copy(dsth_hbm.at[cid, pl.ds(base0, _EB)], dstl[0], sd[0])
        pltpu.async_copy(s_hbm.at[pl.ds(base0, _EB)], cm[0], sc[0])

        def pair(c2, carry):
            for b in (0, 1):
                idx = 2 * c2 + b
                pltpu.make_async_copy(
                    dsth_hbm.at[cid, pl.ds(base0, _EB)], dstl[b], sd[b]).wait()
                pltpu.make_async_copy(
                    s_hbm.at[pl.ds(base0, _EB)], cm[b], sc[b]).wait()
                nidx = jnp.where(idx + 1 == _ECHUNKS, 0, idx + 1)
                nbase = sid * _EPT + nidx * _EB
                pltpu.async_copy(dsth_hbm.at[cid, pl.ds(nbase, _EB)],
                                 dstl[1 - b], sd[1 - b])
                pltpu.async_copy(s_hbm.at[pl.ds(nbase, _EB)],
                                 cm[1 - b], sc[1 - b])
                pltpu.sync_copy(cm[b], acc.at[dstl[b]], add=True)
            return carry
        lax.fori_loop(0, _ECHUNKS // 2, pair, 0)
        pltpu.make_async_copy(
            dsth_hbm.at[cid, pl.ds(base0, _EB)], dstl[0], sd[0]).wait()
        pltpu.make_async_copy(
            s_hbm.at[pl.ds(base0, _EB)], cm[0], sc[0]).wait()
        plsc.subcore_barrier()
        for t in range(_ARPT // _EB):
            r0 = sid * _ARPT + t * _EB
            pltpu.sync_copy(acc.at[pl.ds(r0, _EB)], zb)
            pltpu.sync_copy(zb, out_hbm.at[cid, pl.ds(r0, _EB)])

    return _geom_kernel, _msg_kernel, _scat_kernel


def kernel(x, coords, edge_index, params):
    p = params
    src = edge_index[0].astype(jnp.int32)
    dst = edge_index[1].astype(jnp.int32)
    srcp = jnp.concatenate([src, jnp.zeros((_E_PAD - _E,), jnp.int32)])
    dstp = jnp.concatenate([dst, jnp.full((_E_PAD - _E,), _N_PAD - 1, jnp.int32)])
    coords_p = jnp.pad(coords.astype(_F32), ((0, _N_PAD - _N), (0, 125)))
    xp = jnp.pad(x.astype(_F32), ((0, _N_PAD - _N), (0, 5)))

    enc1, enc2 = p["node_enc"]
    ee1, ee2 = p["edge_enc"]
    dec1, dec2 = p["dec"]

    dsth = jnp.stack([
        jnp.where(dstp < _NHALF, dstp, _AROWS - 1),
        jnp.where(dstp >= _NHALF, dstp - _NHALF, _AROWS - 1),
    ])
    zeros128 = jnp.zeros((_EB, 2 * _HID), _F32)

    _geom_kernel, _msg_kernel, _scat_kernel = _sc_kernels()
    relp = _geom_kernel(coords_p, srcp, dstp)

    we1p = jnp.pad(ee1["W"][:3], ((0, 13), (0, 0)))
    hef = _hef_call(relp, we1p, ee1["W"][3:4], ee1["b"].reshape(1, -1))

    w1p = jnp.pad(enc1["W"], ((0, 5), (0, 0)))
    h = _enc_call(xp, w1p, enc1["b"].reshape(1, -1),
                  enc2["W"], enc2["b"].reshape(1, -1))

    lyr = p["layers"]
    stk = (
        jnp.stack([l["edge_mlp"][0]["W"] for l in lyr]),
        jnp.stack([l["edge_mlp"][0]["b"].reshape(1, -1) for l in lyr]),
        jnp.stack([l["edge_mlp"][1]["W"] for l in lyr]),
        jnp.stack([l["edge_mlp"][1]["b"].reshape(1, -1) for l in lyr]),
        jnp.stack([l["node_mlp"][0]["W"] for l in lyr]),
        jnp.stack([l["node_mlp"][0]["b"].reshape(1, -1) for l in lyr]),
        jnp.stack([l["node_mlp"][1]["W"] for l in lyr]),
        jnp.stack([l["node_mlp"][1]["b"].reshape(1, -1) for l in lyr]),
        jnp.stack([l["ln_g"].reshape(1, -1) for l in lyr]),
        jnp.stack([l["ln_b"].reshape(1, -1) for l in lyr]),
    )
    we2 = ee2["W"]
    be2 = ee2["b"].reshape(1, -1)

    def _step(hc, w):
        em1w, em1b, em2w, em2b, nm1w, nm1b, nm2w, nm2b, lng, lnb = w
        t_t, m_t = _pre_call(hc, em1w[:_HID], em1w[_HID:2 * _HID],
                             em1w[2 * _HID:], we2, em1b, be2)
        cmat = _cmat_call(hef, m_t)
        s_t = _msg_kernel(t_t, cmat, srcp, dstp, zeros128)
        acc2 = _scat_kernel(s_t, dsth, zeros128)
        h2 = _upd_call(acc2, hc, em2w, em2b,
                       nm1w[:_HID], nm1w[_HID:], nm1b,
                       nm2w, nm2b, lng, lnb)
        return h2, jnp.float32(0)

    h, _ = lax.scan(_step, h, stk)

    d2p = jnp.pad(dec2["W"], ((0, 0), (0, 16 - 9)))
    d2bp = jnp.pad(dec2["b"], (0, 16 - 9)).reshape(1, -1)
    out16 = _dec_call(h, dec1["W"], dec1["b"].reshape(1, -1), d2p, d2bp)
    return out16[:_N, :9]
